# Initial kernel scaffold; baseline (speedup 1.0000x reference)
import jax
import jax.numpy as jnp
from jax.experimental import pallas as pl


def kernel(query_x, anchor_x):
    raise NotImplementedError("write your pallas kernel here")


# speedup vs baseline: 7.6624x; 7.6624x over previous
import functools

import jax
import jax.numpy as jnp
from jax import lax
from jax.experimental import pallas as pl
from jax.experimental.pallas import tpu as pltpu
from jax.experimental.pallas import tpu_sc as plsc

K_NN = 32
QB = 128
NB = 16384
CH = 128
QT = 8
NC = 2
NS = 16


def _k1_body(n_real, q_ref, a_ref, d2_ref, m_ref):
    t = pl.program_id(0)
    q = q_ref[...]
    a = a_ref[...]
    dn = (((1,), (1,)), ((), ()))
    qa = lax.dot_general(q, a, dn, preferred_element_type=jnp.float32)
    ones = jnp.ones((8, a.shape[1]), jnp.float32)
    a2 = lax.dot_general(ones, a * a, dn, precision=lax.Precision.HIGHEST,
                         preferred_element_type=jnp.float32)
    q2 = jnp.sum(q * q, axis=1, keepdims=True)
    d2 = (q2 + a2[0:1, :]) - 2.0 * qa
    colg = t * NB + lax.broadcasted_iota(jnp.int32, (1, NB), 1)
    d2 = jnp.where(colg < n_real, d2, jnp.inf)
    for c in range(NB // CH):
        blk = d2[:, c * CH:(c + 1) * CH]
        d2_ref[:, c, :, :] = blk.reshape(QB // QT, QT, CH)
        m_ref[:, c:c + 1] = jnp.min(blk, axis=1, keepdims=True)


def _k2_body(n_chunks, m_ref, tid_ref, gidx_ref):
    qb = pl.program_id(0)
    iota = lax.broadcasted_iota(jnp.int32, (QB, n_chunks), 1)
    lane = lax.broadcasted_iota(jnp.int32, (1, CH), 1)
    rowq = qb * QB + lax.broadcasted_iota(jnp.int32, (QB, 1), 0)
    rowt = (rowq // QT) * n_chunks

    kiota = lax.broadcasted_iota(jnp.int32, (1, K_NN), 1)

    def step(k, carry):
        vals, tids = carry
        mn = jnp.min(vals, axis=1, keepdims=True)
        eq = vals == mn
        cid = jnp.min(jnp.where(eq, iota, n_chunks), axis=1, keepdims=True)
        tids = jnp.where(kiota == k, rowt + cid, tids)
        gidx_ref[:, pl.ds(pl.multiple_of(k * CH, CH), CH)] = cid * CH + lane
        return jnp.where(iota == cid, jnp.inf, vals), tids

    _, tids = lax.fori_loop(
        0, K_NN, step,
        (m_ref[...], jnp.zeros((QB, K_NN), jnp.int32)))
    tid_ref[...] = tids


def _k4_body(g_ref, gidx_ref, dist_ref, idx_ref):
    gidx = gidx_ref[...]
    big = jnp.int32(2 ** 30)
    kiota = lax.broadcasted_iota(jnp.int32, (1, K_NN), 1)

    def step(k, carry):
        vals, dists, idxs = carry
        mn = jnp.min(vals, axis=1, keepdims=True)
        eq = vals == mn
        gsel = jnp.min(jnp.where(eq, gidx, big), axis=1, keepdims=True)
        dists = jnp.where(kiota == k, mn, dists)
        idxs = jnp.where(kiota == k, gsel, idxs)
        return jnp.where(gidx == gsel, jnp.inf, vals), dists, idxs

    _, dists, idxs = lax.fori_loop(
        0, K_NN, step,
        (g_ref[...], jnp.zeros((QB, K_NN), jnp.float32),
         jnp.zeros((QB, K_NN), jnp.int32)))
    dist_ref[...] = dists
    idx_ref[...] = idxs


def _make_sc_gather(n_rows_table, n_ids):
    per_w = n_ids // (NC * NS)
    n_batches = per_w // K_NN
    mesh = plsc.VectorSubcoreMesh(core_axis_name="c", subcore_axis_name="s")

    @functools.partial(
        pl.kernel, mesh=mesh,
        out_type=jax.ShapeDtypeStruct((n_ids, CH), jnp.float32),
        scratch_types=[
            pltpu.VMEM((K_NN,), jnp.int32),
            pltpu.VMEM((K_NN, QT, CH), jnp.float32),
            pltpu.VMEM((K_NN, CH), jnp.float32),
            pltpu.SemaphoreType.DMA,
        ],
    )
    def gather_k(table_hbm, ids_hbm, out_hbm, idx_v, rows_v, out_v, sem):
        wid = lax.axis_index("s") * NC + lax.axis_index("c")
        base = wid * per_w

        def body(j, carry):
            off = base + j * K_NN
            s = j % QT
            pltpu.sync_copy(ids_hbm.at[pl.ds(off, K_NN)], idx_v)
            pltpu.async_copy(table_hbm.at[idx_v], rows_v, sem).wait()

            def pick(i, c2):
                for kk in range(CH // 16):
                    out_v[i, pl.ds(kk * 16, 16)] = (
                        rows_v[i, s, pl.ds(kk * 16, 16)])
                return c2

            lax.fori_loop(0, K_NN, pick, 0)
            pltpu.sync_copy(out_v, out_hbm.at[pl.ds(off, K_NN)])
            return carry

        lax.fori_loop(0, n_batches, body, 0)

    return gather_k


def kernel(query_x, anchor_x):
    qn, d = query_x.shape
    n = anchor_x.shape[0]
    n_tiles = -(-n // NB)
    n_pad = n_tiles * NB
    n_chunks = n_pad // CH
    n_qb = qn // QB

    a_pad = jnp.pad(anchor_x, ((0, n_pad - n), (0, 0)))

    d2, m = pl.pallas_call(
        functools.partial(_k1_body, n),
        grid=(n_tiles, n_qb),
        in_specs=[
            pl.BlockSpec((QB, d), lambda t, qb: (qb, 0)),
            pl.BlockSpec((NB, d), lambda t, qb: (t, 0)),
        ],
        out_specs=[
            pl.BlockSpec((QB // QT, NB // CH, QT, CH),
                         lambda t, qb: (qb, t, 0, 0)),
            pl.BlockSpec((QB, NB // CH), lambda t, qb: (qb, t)),
        ],
        out_shape=[
            jax.ShapeDtypeStruct((qn // QT, n_chunks, QT, CH), jnp.float32),
            jax.ShapeDtypeStruct((qn, n_chunks), jnp.float32),
        ],
        compiler_params=pltpu.CompilerParams(
            dimension_semantics=("arbitrary", "arbitrary")),
    )(query_x, a_pad)

    tid, gidx = pl.pallas_call(
        functools.partial(_k2_body, n_chunks),
        grid=(n_qb,),
        in_specs=[pl.BlockSpec((QB, n_chunks), lambda qb: (qb, 0))],
        out_specs=[
            pl.BlockSpec((QB, K_NN), lambda qb: (qb, 0)),
            pl.BlockSpec((QB, K_NN * CH), lambda qb: (qb, 0)),
        ],
        out_shape=[
            jax.ShapeDtypeStruct((qn, K_NN), jnp.int32),
            jax.ShapeDtypeStruct((qn, K_NN * CH), jnp.int32),
        ],
    )(m)

    table = d2.reshape((qn // QT) * n_chunks, QT, CH)
    ids = tid.reshape(qn * K_NN)
    g = _make_sc_gather((qn // QT) * n_chunks, qn * K_NN)(table, ids)
    g2 = g.reshape(qn, K_NN * CH)

    nn_dist, nn_idx = pl.pallas_call(
        _k4_body,
        grid=(n_qb,),
        in_specs=[
            pl.BlockSpec((QB, K_NN * CH), lambda qb: (qb, 0)),
            pl.BlockSpec((QB, K_NN * CH), lambda qb: (qb, 0)),
        ],
        out_specs=[
            pl.BlockSpec((QB, K_NN), lambda qb: (qb, 0)),
            pl.BlockSpec((QB, K_NN), lambda qb: (qb, 0)),
        ],
        out_shape=[
            jax.ShapeDtypeStruct((qn, K_NN), jnp.float32),
            jax.ShapeDtypeStruct((qn, K_NN), jnp.int32),
        ],
    )(g2, gidx)

    return (nn_dist, nn_idx)


# speedup vs baseline: 8.0970x; 1.0567x over previous
import functools

import jax
import jax.numpy as jnp
from jax import lax
from jax.experimental import pallas as pl
from jax.experimental.pallas import tpu as pltpu
from jax.experimental.pallas import tpu_sc as plsc

K_NN = 32
QB = 128
NB = 16384
CH = 128
QT = 8
NC = 2
NS = 16


def _k1_body(n_real, q_ref, a_ref, d2_ref, m_ref):
    t = pl.program_id(0)
    q = q_ref[...]
    a = a_ref[...]
    dn = (((1,), (1,)), ((), ()))
    qa = lax.dot_general(q, a, dn, preferred_element_type=jnp.float32)
    ones = jnp.ones((8, a.shape[1]), jnp.float32)
    a2 = lax.dot_general(ones, a * a, dn, precision=lax.Precision.HIGHEST,
                         preferred_element_type=jnp.float32)
    q2 = jnp.sum(q * q, axis=1, keepdims=True)
    d2 = (q2 + a2[0:1, :]) - 2.0 * qa
    colg = t * NB + lax.broadcasted_iota(jnp.int32, (1, NB), 1)
    d2 = jnp.where(colg < n_real, d2, jnp.inf)
    for c in range(NB // CH):
        blk = d2[:, c * CH:(c + 1) * CH]
        d2_ref[:, c, :, :] = blk.reshape(QB // QT, QT, CH)
        m_ref[:, c:c + 1] = jnp.min(blk, axis=1, keepdims=True)


def _k2_body(n_chunks, m_ref, tid_ref, gidx_ref):
    qb = pl.program_id(0)
    iota = lax.broadcasted_iota(jnp.int32, (QB, n_chunks), 1)
    lane = lax.broadcasted_iota(jnp.int32, (1, CH), 1)
    rowq = qb * QB + lax.broadcasted_iota(jnp.int32, (QB, 1), 0)
    rowt = (rowq // QT) * n_chunks

    kiota = lax.broadcasted_iota(jnp.int32, (1, K_NN), 1)

    def step(k, carry):
        vals, tids = carry
        mn = jnp.min(vals, axis=1, keepdims=True)
        eq = vals == mn
        cid = jnp.min(jnp.where(eq, iota, n_chunks), axis=1, keepdims=True)
        tids = jnp.where(kiota == k, rowt + cid, tids)
        gidx_ref[:, pl.ds(pl.multiple_of(k * CH, CH), CH)] = cid * CH + lane
        return jnp.where(iota == cid, jnp.inf, vals), tids

    _, tids = lax.fori_loop(
        0, K_NN, step,
        (m_ref[...], jnp.zeros((QB, K_NN), jnp.int32)))
    tid_ref[...] = tids


def _k4_body(g_ref, gidx_ref, dist_ref, idx_ref):
    gidx = gidx_ref[...]
    big = jnp.int32(2 ** 30)
    kiota = lax.broadcasted_iota(jnp.int32, (1, K_NN), 1)

    def step(k, carry):
        vals, dists, idxs = carry
        mn = jnp.min(vals, axis=1, keepdims=True)
        eq = vals == mn
        gsel = jnp.min(jnp.where(eq, gidx, big), axis=1, keepdims=True)
        dists = jnp.where(kiota == k, mn, dists)
        idxs = jnp.where(kiota == k, gsel, idxs)
        return jnp.where(gidx == gsel, jnp.inf, vals), dists, idxs

    _, dists, idxs = lax.fori_loop(
        0, K_NN, step,
        (g_ref[...], jnp.zeros((QB, K_NN), jnp.float32),
         jnp.zeros((QB, K_NN), jnp.int32)))
    dist_ref[...] = dists
    idx_ref[...] = idxs


def _make_sc_gather(n_rows_table, n_ids):
    per_w = n_ids // (NC * NS)
    n_batches = per_w // K_NN
    mesh = plsc.VectorSubcoreMesh(core_axis_name="c", subcore_axis_name="s")

    @functools.partial(
        pl.kernel, mesh=mesh,
        out_type=jax.ShapeDtypeStruct((n_ids, CH), jnp.float32),
        scratch_types=[
            pltpu.VMEM((K_NN,), jnp.int32),
            pltpu.VMEM((K_NN,), jnp.int32),
            pltpu.VMEM((K_NN, QT, CH), jnp.float32),
            pltpu.VMEM((K_NN, QT, CH), jnp.float32),
            pltpu.VMEM((K_NN, CH), jnp.float32),
            pltpu.SemaphoreType.DMA,
            pltpu.SemaphoreType.DMA,
        ],
    )
    def gather_k(table_hbm, ids_hbm, out_hbm, idx_v0, idx_v1, rows_v0,
                 rows_v1, out_v, sem0, sem1):
        wid = lax.axis_index("s") * NC + lax.axis_index("c")
        base = wid * per_w

        def start(j, idx_v, rows_v, sem):
            off = base + j * K_NN
            pltpu.sync_copy(ids_hbm.at[pl.ds(off, K_NN)], idx_v)
            pltpu.async_copy(table_hbm.at[idx_v], rows_v, sem)

        def finish(j, idx_v, rows_v, sem):
            pltpu.make_async_copy(table_hbm.at[idx_v], rows_v, sem).wait()
            s = j % QT

            def pick(i, c2):
                for kk in range(CH // 16):
                    out_v[i, pl.ds(kk * 16, 16)] = (
                        rows_v[i, s, pl.ds(kk * 16, 16)])
                return c2

            lax.fori_loop(0, K_NN, pick, 0)
            pltpu.sync_copy(out_v, out_hbm.at[pl.ds(base + j * K_NN, K_NN)])

        start(0, idx_v0, rows_v0, sem0)

        def body(i, carry):
            j0 = 2 * i
            start(j0 + 1, idx_v1, rows_v1, sem1)
            finish(j0, idx_v0, rows_v0, sem0)
            start(jnp.minimum(j0 + 2, n_batches - 1), idx_v0, rows_v0, sem0)
            finish(j0 + 1, idx_v1, rows_v1, sem1)
            return carry

        lax.fori_loop(0, n_batches // 2, body, 0)
        pltpu.make_async_copy(table_hbm.at[idx_v0], rows_v0, sem0).wait()

    return gather_k


def kernel(query_x, anchor_x):
    qn, d = query_x.shape
    n = anchor_x.shape[0]
    n_tiles = -(-n // NB)
    n_pad = n_tiles * NB
    n_chunks = n_pad // CH
    n_qb = qn // QB

    a_pad = jnp.pad(anchor_x, ((0, n_pad - n), (0, 0)))

    d2, m = pl.pallas_call(
        functools.partial(_k1_body, n),
        grid=(n_tiles, n_qb),
        in_specs=[
            pl.BlockSpec((QB, d), lambda t, qb: (qb, 0)),
            pl.BlockSpec((NB, d), lambda t, qb: (t, 0)),
        ],
        out_specs=[
            pl.BlockSpec((QB // QT, NB // CH, QT, CH),
                         lambda t, qb: (qb, t, 0, 0)),
            pl.BlockSpec((QB, NB // CH), lambda t, qb: (qb, t)),
        ],
        out_shape=[
            jax.ShapeDtypeStruct((qn // QT, n_chunks, QT, CH), jnp.float32),
            jax.ShapeDtypeStruct((qn, n_chunks), jnp.float32),
        ],
        compiler_params=pltpu.CompilerParams(
            dimension_semantics=("arbitrary", "arbitrary")),
    )(query_x, a_pad)

    tid, gidx = pl.pallas_call(
        functools.partial(_k2_body, n_chunks),
        grid=(n_qb,),
        in_specs=[pl.BlockSpec((QB, n_chunks), lambda qb: (qb, 0))],
        out_specs=[
            pl.BlockSpec((QB, K_NN), lambda qb: (qb, 0)),
            pl.BlockSpec((QB, K_NN * CH), lambda qb: (qb, 0)),
        ],
        out_shape=[
            jax.ShapeDtypeStruct((qn, K_NN), jnp.int32),
            jax.ShapeDtypeStruct((qn, K_NN * CH), jnp.int32),
        ],
    )(m)

    table = d2.reshape((qn // QT) * n_chunks, QT, CH)
    ids = tid.reshape(qn * K_NN)
    g = _make_sc_gather((qn // QT) * n_chunks, qn * K_NN)(table, ids)
    g2 = g.reshape(qn, K_NN * CH)

    nn_dist, nn_idx = pl.pallas_call(
        _k4_body,
        grid=(n_qb,),
        in_specs=[
            pl.BlockSpec((QB, K_NN * CH), lambda qb: (qb, 0)),
            pl.BlockSpec((QB, K_NN * CH), lambda qb: (qb, 0)),
        ],
        out_specs=[
            pl.BlockSpec((QB, K_NN), lambda qb: (qb, 0)),
            pl.BlockSpec((QB, K_NN), lambda qb: (qb, 0)),
        ],
        out_shape=[
            jax.ShapeDtypeStruct((qn, K_NN), jnp.float32),
            jax.ShapeDtypeStruct((qn, K_NN), jnp.int32),
        ],
    )(g2, gidx)

    return (nn_dist, nn_idx)
